# Initial kernel scaffold; baseline (speedup 1.0000x reference)
import jax
import jax.numpy as jnp
from jax.experimental import pallas as pl


def kernel(xyzs, feats, params):
    raise NotImplementedError("write your pallas kernel here")


# speedup vs baseline: 17.1886x; 17.1886x over previous
import functools
import math

import jax
import jax.numpy as jnp
from jax import lax
from jax.experimental import pallas as pl
from jax.experimental.pallas import tpu as pltpu
from jax.experimental.pallas import tpu_sc as plsc

B = 4
N = 8192
DIM = 128
HID = 64
NG = 8
K = 16
M = 2048

NEG = -3.4028235e38
F32 = jnp.float32
I32 = jnp.int32

NC = 2
NS = 16
NW = NC * NS


def _fps_body(x_ref, y_ref, z_ref, si_ref, sx_ref, sy_ref, sz_ref):
    x = x_ref[...]
    y = y_ref[...]
    z = z_ref[...]
    ion = lax.broadcasted_iota(I32, (B, N), 1)
    iom = lax.broadcasted_iota(I32, (B, M), 1)

    def body(i, carry):
        dists, far = carry
        sel = ion == far
        cx = jnp.sum(jnp.where(sel, x, 0.0), axis=1, keepdims=True)
        cy = jnp.sum(jnp.where(sel, y, 0.0), axis=1, keepdims=True)
        cz = jnp.sum(jnp.where(sel, z, 0.0), axis=1, keepdims=True)
        lm = iom == i
        si_ref[...] = jnp.where(lm, jnp.broadcast_to(far, (B, M)), si_ref[...])
        sx_ref[...] = jnp.where(lm, jnp.broadcast_to(cx, (B, M)), sx_ref[...])
        sy_ref[...] = jnp.where(lm, jnp.broadcast_to(cy, (B, M)), sy_ref[...])
        sz_ref[...] = jnp.where(lm, jnp.broadcast_to(cz, (B, M)), sz_ref[...])
        dx = x - cx
        dy = y - cy
        dz = z - cz
        d = dx * dx + dy * dy + dz * dz
        dists = jnp.minimum(dists, d)
        mx = jnp.max(dists, axis=1, keepdims=True)
        far2 = jnp.min(jnp.where(dists == mx, ion, N), axis=1, keepdims=True)
        return dists, far2.astype(I32)

    d0 = jnp.full((B, N), 1e10, F32)
    f0 = jnp.zeros((B, 1), I32)
    lax.fori_loop(0, M, body, (d0, f0))


def _fps(x_bn, y_bn, z_bn):
    return pl.pallas_call(
        _fps_body,
        out_shape=[
            jax.ShapeDtypeStruct((B, M), I32),
            jax.ShapeDtypeStruct((B, M), F32),
            jax.ShapeDtypeStruct((B, M), F32),
            jax.ShapeDtypeStruct((B, M), F32),
        ],
    )(x_bn, y_bn, z_bn)


NT1 = 512


def _o2s_body(xc_ref, yc_ref, zc_ref, sx_ref, sy_ref, sz_ref, o_ref):
    t = pl.program_id(0)
    b = t // (N // NT1)
    subl = lax.broadcasted_iota(I32, (B, M), 0)

    def pick(r):
        return jnp.sum(jnp.where(subl == b, r[...], 0.0), axis=0, keepdims=True)

    sx = pick(sx_ref)
    sy = pick(sy_ref)
    sz = pick(sz_ref)
    qx = xc_ref[...]
    qy = yc_ref[...]
    qz = zc_ref[...]
    bf = lambda v: v.astype(jnp.bfloat16).astype(F32)
    dot = bf(qx) * bf(sx) + bf(qy) * bf(sy) + bf(qz) * bf(sz)
    qq = qx * qx + qy * qy + qz * qz
    ss = sx * sx + sy * sy + sz * sz
    d = (qq + ss) - 2.0 * dot
    mn = jnp.min(d, axis=1, keepdims=True)
    iom = lax.broadcasted_iota(I32, (NT1, M), 1)
    idx = jnp.min(jnp.where(d == mn, iom, M), axis=1, keepdims=True)
    o_ref[...] = idx


def _o2s(x_c, y_c, z_c, sx, sy, sz):
    col = pl.BlockSpec((NT1, 1), lambda t: (t, 0))
    full = pl.BlockSpec((B, M), lambda t: (0, 0))
    return pl.pallas_call(
        _o2s_body,
        grid=(B * N // NT1,),
        in_specs=[col, col, col, full, full, full],
        out_specs=pl.BlockSpec((NT1, 1), lambda t: (t, 0)),
        out_shape=jax.ShapeDtypeStruct((B * N, 1), I32),
    )(x_c, y_c, z_c, sx, sy, sz)


MT2 = 256


def _knn_body(sx_ref, sy_ref, sz_ref, x_ref, y_ref, z_ref, o_ref):
    t = pl.program_id(0)
    b = t // (M // MT2)
    subl = lax.broadcasted_iota(I32, (B, N), 0)

    def pick(r):
        return jnp.sum(jnp.where(subl == b, r[...], 0.0), axis=0, keepdims=True)

    qx = sx_ref[...]
    qy = sy_ref[...]
    qz = sz_ref[...]
    x = pick(x_ref)
    y = pick(y_ref)
    z = pick(z_ref)
    bf = lambda v: v.astype(jnp.bfloat16).astype(F32)
    dot = bf(qx) * bf(x) + bf(qy) * bf(y) + bf(qz) * bf(z)
    qq = qx * qx + qy * qy + qz * qz
    ss = x * x + y * y + z * z
    d = (qq + ss) - 2.0 * dot
    ion = lax.broadcasted_iota(I32, (MT2, N), 1)
    for k in range(K):
        mn = jnp.min(d, axis=1, keepdims=True)
        idx = jnp.min(jnp.where(d == mn, ion, N), axis=1, keepdims=True)
        o_ref[:, :, k : k + 1] = idx.reshape(1, MT2, 1)
        d = jnp.where(ion == idx, jnp.inf, d)


def _knn(sx_c, sy_c, sz_c, x_bn, y_bn, z_bn):
    col = pl.BlockSpec((MT2, 1), lambda t: (t, 0))
    full = pl.BlockSpec((B, N), lambda t: (0, 0))
    tpb = M // MT2
    return pl.pallas_call(
        _knn_body,
        grid=(B * M // MT2,),
        in_specs=[col, col, col, full, full, full],
        out_specs=pl.BlockSpec((1, MT2, K), lambda t: (t // tpb, t % tpb, 0)),
        out_shape=jax.ShapeDtypeStruct((B, M, K), I32),
    )(sx_c, sy_c, sz_c, x_bn, y_bn, z_bn)


RT = 2048


def _convs_body(x_ref, wp_ref, bp_ref, wkv_ref, bkv_ref, f_ref, kv_ref):
    x = x_ref[...]
    f = jnp.dot(x, wp_ref[...], preferred_element_type=F32) + bp_ref[...]
    f_ref[...] = f
    kv_ref[...] = jnp.dot(f, wkv_ref[...], preferred_element_type=F32) + bkv_ref[...]


def _convs(feats_rows, wpT, bp, wkvT, bkv):
    full = lambda a: pl.BlockSpec(a.shape, lambda t: (0,) * a.ndim)
    return pl.pallas_call(
        _convs_body,
        grid=(B * N // RT,),
        in_specs=[
            pl.BlockSpec((RT, DIM), lambda t: (t, 0)),
            full(wpT), full(bp), full(wkvT), full(bkv),
        ],
        out_specs=[
            pl.BlockSpec((RT, DIM), lambda t: (t, 0)),
            pl.BlockSpec((RT, DIM), lambda t: (t, 0)),
        ],
        out_shape=[
            jax.ShapeDtypeStruct((B * N, DIM), F32),
            jax.ShapeDtypeStruct((B * N, DIM), F32),
        ],
    )(feats_rows, wpT, bp, wkvT, bkv)


KR = B * M * K
SR = B * M
OR = B * N
KR_W = KR // NW
SR_W = SR // NW
OR_W = OR // NW


def _sc_body(tkv, tmo, tf, tfeat, kidx2, sidx2, oidx2,
             kv_g, mo_g, sfeat_g, ident_g, dn2,
             idx_v, sidx_v, oidx_v, buf_kv, buf_mo,
             buf_f, buf_ft, ones_v, zb, shared, sem):
    cid = lax.axis_index("c")
    sid = lax.axis_index("s")
    wid = sid * NC + cid

    pltpu.sync_copy(kidx2.at[pl.ds(wid * 32, 32)], idx_v)
    pltpu.sync_copy(sidx2.at[pl.ds(wid * 2, 2)], sidx_v)
    pltpu.sync_copy(oidx2.at[pl.ds(wid * 8, 8)], oidx_v)

    for i in range(8):
        zb[pl.ds(i * 16, 16)] = jnp.zeros((16,), F32)
        ones_v[pl.ds(i * 16, 16)] = jnp.ones((16,), F32)

    for j in range(4):
        pltpu.sync_copy(zb, shared.at[pl.ds(sid * 512 + j * 128, 128)])
    plsc.subcore_barrier()

    def obody(j, c):
        pltpu.sync_copy(ones_v, shared.at[oidx_v.at[j]], add=True)
        return c

    lax.fori_loop(0, 8, obody, 0)
    plsc.subcore_barrier()

    @pl.when(sid == 0)
    def _():
        pltpu.sync_copy(shared, dn2.at[cid])

    def gbody(j, c):
        row = idx_v.at[j]
        base = wid * KR_W + j * 128
        pltpu.async_copy(tkv.at[row], buf_kv, sem).wait()
        pltpu.sync_copy(buf_kv, kv_g.at[pl.ds(base, 128)])
        pltpu.async_copy(tmo.at[row], buf_mo, sem).wait()
        pltpu.sync_copy(buf_mo, mo_g.at[pl.ds(base, 128)])
        return c

    lax.fori_loop(0, 32, gbody, 0)

    def sbody(j, c):
        row = sidx_v.at[j]
        base = wid * SR_W + j * 128
        pltpu.async_copy(tf.at[row], buf_f, sem).wait()
        pltpu.sync_copy(buf_f, sfeat_g.at[pl.ds(base, 128)])
        pltpu.async_copy(tfeat.at[row], buf_ft, sem).wait()
        pltpu.sync_copy(buf_ft, ident_g.at[pl.ds(base, 128)])
        return c

    lax.fori_loop(0, 2, sbody, 0)


def _sc_gathers(tkv, tmo, tf, tfeat, kidx2, sidx2, oidx2):
    mesh = plsc.VectorSubcoreMesh(core_axis_name="c", subcore_axis_name="s")
    return pl.kernel(
        _sc_body,
        out_type=[
            jax.ShapeDtypeStruct((KR, DIM), F32),
            jax.ShapeDtypeStruct((KR, DIM), F32),
            jax.ShapeDtypeStruct((SR, DIM), F32),
            jax.ShapeDtypeStruct((SR, DIM), F32),
            jax.ShapeDtypeStruct((NC, SR), F32),
        ],
        mesh=mesh,
        compiler_params=pltpu.CompilerParams(needs_layout_passes=False),
        scratch_types=[
            pltpu.VMEM((32, 128), I32),
            pltpu.VMEM((2, 128), I32),
            pltpu.VMEM((8, 128), I32),
            pltpu.VMEM((128, DIM), F32),
            pltpu.VMEM((128, DIM), F32),
            pltpu.VMEM((128, DIM), F32),
            pltpu.VMEM((128, DIM), F32),
            pltpu.VMEM((128,), F32),
            pltpu.VMEM((128,), F32),
            pltpu.VMEM_SHARED((SR,), F32),
            pltpu.SemaphoreType.DMA,
        ],
    )(tkv, tmo, tf, tfeat, kidx2, sidx2, oidx2)


def _gmat():
    ci = lax.broadcasted_iota(I32, (HID, NG), 0) // (HID // NG)
    gi = lax.broadcasted_iota(I32, (HID, NG), 1)
    return (ci == gi).astype(F32)


def _emat():
    ci = lax.broadcasted_iota(I32, (NG, HID), 1) // (HID // NG)
    gi = lax.broadcasted_iota(I32, (NG, HID), 0)
    return (ci == gi).astype(F32)


def _gn_accum(stats_ref, r0, h):
    g = _gmat()
    s = jnp.dot(jnp.sum(h, axis=0, keepdims=True), g, preferred_element_type=F32)
    sq = jnp.dot(jnp.sum(h * h, axis=0, keepdims=True), g, preferred_element_type=F32)
    stats_ref[r0 : r0 + 1, 0:NG] += s
    stats_ref[r0 + 1 : r0 + 2, 0:NG] += sq


def _gn_apply(stats_ref, r0, cnt, h, gam_ref, bet_ref):
    e = _emat()
    s = stats_ref[r0 : r0 + 1, 0:NG]
    sq = stats_ref[r0 + 1 : r0 + 2, 0:NG]
    mean = s / cnt
    var = sq / cnt - mean * mean
    rstd = 1.0 / jnp.sqrt(var + 1e-5)
    ml = jnp.dot(mean, e, preferred_element_type=F32)
    rl = jnp.dot(rstd, e, preferred_element_type=F32)
    return (h - ml) * rl * gam_ref[...] + bet_ref[...]


MTA = 256
MTN = M // MTA
CNT_A = (HID // NG) * M * K


def _attn_body(kv_ref, mo_ref, sfeat_ref, smisc_ref,
               dn_ref, wq_ref, bq_ref, cd1_ref, cd1b_ref, gn1w_ref, gn1b_ref,
               cd2_ref, cd2b_ref, cg1_ref, cg1b_ref, gn2w_ref, gn2b_ref,
               cg2_ref, cg2b_ref, post_ref, postb_ref,
               anc_ref, md_ref, stats_ref):
    p = pl.program_id(1)
    t = pl.program_id(2)

    @pl.when((p == 0) & (t == 0))
    def _():
        stats_ref[...] = jnp.zeros_like(stats_ref)

    mo = mo_ref[...]
    misc = mo[:, 0:16]
    smisc = smisc_ref[...]
    s3 = jnp.broadcast_to(smisc[:, None, :], (MTA, K, 16)).reshape(MTA * K, 16)
    pos = s3 - misc
    h1 = jnp.dot(pos, cd1_ref[...], preferred_element_type=F32) + cd1b_ref[...]

    @pl.when(p == 0)
    def _():
        _gn_accum(stats_ref, 0, h1)

    @pl.when(p > 0)
    def _():
        h1n = jnp.maximum(_gn_apply(stats_ref, 0, CNT_A, h1, gn1w_ref, gn1b_ref), 0.0)
        pos_enc = jnp.dot(h1n, cd2_ref[...], preferred_element_type=F32) + cd2b_ref[...]
        q = jnp.dot(sfeat_ref[...], wq_ref[...], preferred_element_type=F32) + bq_ref[...]
        kv = kv_ref[...]
        key3 = kv[:, 0:HID].reshape(MTA, K, HID)
        pe3 = pos_enc.reshape(MTA, K, HID)
        a0 = (q[:, None, :] - key3 + pe3).reshape(MTA * K, HID)
        a1 = jnp.dot(a0, cg1_ref[...], preferred_element_type=F32) + cg1b_ref[...]

        @pl.when(p == 1)
        def _():
            _gn_accum(stats_ref, 2, a1)

        @pl.when(p == 2)
        def _():
            a1n = jnp.maximum(_gn_apply(stats_ref, 2, CNT_A, a1, gn2w_ref, gn2b_ref), 0.0)
            a2 = (jnp.dot(a1n, cg2_ref[...], preferred_element_type=F32) + cg2b_ref[...])
            a2 = a2 * (1.0 / math.sqrt(HID))
            rowm = t * MTA + lax.div(lax.broadcasted_iota(I32, (MTA * K, 1), 0), K)
            maskr = mo[:, 3:4] == rowm.astype(F32)
            a3 = jnp.where(maskr, a2, NEG).reshape(MTA, K, HID)
            mx = jnp.max(a3, axis=1, keepdims=True)
            ex = jnp.exp(a3 - mx)
            soft = ex / jnp.sum(ex, axis=1, keepdims=True)
            v3 = kv[:, HID:DIM].reshape(MTA, K, HID) + pe3
            res = jnp.sum(soft * v3, axis=1)
            anc_ref[...] = (jnp.dot(res, post_ref[...], preferred_element_type=F32)
                            + postb_ref[...] + sfeat_ref[...])[None]
            lane16 = lax.broadcasted_iota(I32, (MTA * K, 16), 1)
            diff = jnp.where(lane16 < 3, misc - s3, 0.0)
            sq = jnp.sum(diff * diff, axis=1, keepdims=True) + 1e-12
            dist = jnp.sqrt(sq)
            dsum = jnp.sum(jnp.where(maskr, dist, 0.0).reshape(MTA, K, 1), axis=1)
            dn = dn_ref[:, 0:1] + dn_ref[:, 1:2]
            md_ref[...] = (dsum / dn)[None]


def _attn(kv_g, mo_g, sfeat_g, smisc, dn2t, params_t):
    (wqT, bq, cd1T, cd1b, gn1w, gn1b, cd2T, cd2b, cg1T, cg1b, gn2w, gn2b,
     cg2T, cg2b, postT, postb) = params_t
    rowsK = lambda c: pl.BlockSpec((MTA * K, c), lambda b, p, t: (b * MTN + t, 0))
    rowsM = lambda c: pl.BlockSpec((MTA, c), lambda b, p, t: (b * MTN + t, 0))
    full = lambda a: pl.BlockSpec(a.shape, lambda b, p, t: (0,) * a.ndim)
    return pl.pallas_call(
        _attn_body,
        grid=(B, 3, MTN),
        in_specs=[
            rowsK(DIM), rowsK(DIM), rowsM(DIM), rowsM(16), rowsM(2),
            full(wqT), full(bq), full(cd1T), full(cd1b), full(gn1w), full(gn1b),
            full(cd2T), full(cd2b), full(cg1T), full(cg1b), full(gn2w), full(gn2b),
            full(cg2T), full(cg2b), full(postT), full(postb),
        ],
        out_specs=[
            pl.BlockSpec((1, MTA, DIM), lambda b, p, t: (p, b * MTN + t, 0)),
            pl.BlockSpec((1, MTA, 1), lambda b, p, t: (p, b * MTN + t, 0)),
        ],
        out_shape=[
            jax.ShapeDtypeStruct((3, SR, DIM), F32),
            jax.ShapeDtypeStruct((3, SR, 1), F32),
        ],
        scratch_shapes=[pltpu.VMEM((8, 128), F32)],
    )(kv_g, mo_g, sfeat_g, smisc, dn2t, *params_t)


def _pos_body(mo_ref, smisc_ref,
              pe1_ref, pe1b_ref, gn3w_ref, gn3b_ref, pe2_ref, pe2b_ref,
              pa1_ref, pa1b_ref, gn4w_ref, gn4b_ref, pa2_ref, pa2b_ref,
              out_ref, stats_ref):
    p = pl.program_id(1)
    t = pl.program_id(2)

    @pl.when((p == 0) & (t == 0))
    def _():
        stats_ref[...] = jnp.zeros_like(stats_ref)

    mo = mo_ref[...]
    misc = mo[:, 0:16]
    smisc = smisc_ref[...]
    s3 = jnp.broadcast_to(smisc[:, None, :], (MTA, K, 16)).reshape(MTA * K, 16)
    lane = lax.broadcasted_iota(I32, (MTA * K, 16), 1)
    diff = jnp.where(lane < 3, misc - s3, 0.0)
    sq = jnp.sum(diff * diff, axis=1, keepdims=True) + 1e-12
    d = jnp.sqrt(sq)
    direction = diff / jnp.maximum(d, 1e-12)
    local = direction + jnp.where(lane == 3, d, 0.0)
    h = jnp.dot(local, pe1_ref[...], preferred_element_type=F32) + pe1b_ref[...]

    @pl.when(p == 0)
    def _():
        _gn_accum(stats_ref, 0, h)

    @pl.when(p > 0)
    def _():
        hn = jnp.maximum(_gn_apply(stats_ref, 0, CNT_A, h, gn3w_ref, gn3b_ref), 0.0)
        pe = jnp.dot(hn, pe2_ref[...], preferred_element_type=F32) + pe2b_ref[...]
        a2p = jnp.dot(pe, pa1_ref[...], preferred_element_type=F32) + pa1b_ref[...]

        @pl.when(p == 1)
        def _():
            _gn_accum(stats_ref, 2, a2p)

        @pl.when(p == 2)
        def _():
            a2n = jnp.maximum(_gn_apply(stats_ref, 2, CNT_A, a2p, gn4w_ref, gn4b_ref), 0.0)
            a2 = jnp.dot(a2n, pa2_ref[...], preferred_element_type=F32) + pa2b_ref[...]
            rowm = t * MTA + lax.div(lax.broadcasted_iota(I32, (MTA * K, 1), 0), K)
            maskr = mo[:, 3:4] == rowm.astype(F32)
            a3 = jnp.where(maskr, a2, NEG).reshape(MTA, K, DIM)
            mx = jnp.max(a3, axis=1, keepdims=True)
            ex = jnp.exp(a3 - mx)
            soft = ex / jnp.sum(ex, axis=1, keepdims=True)
            pe3 = pe.reshape(MTA, K, DIM)
            out_ref[...] = jnp.sum(soft * pe3, axis=1)[None]


def _pos_emb(mo_g, smisc, params_t):
    rowsK = lambda c: pl.BlockSpec((MTA * K, c), lambda b, p, t: (b * MTN + t, 0))
    rowsM = lambda c: pl.BlockSpec((MTA, c), lambda b, p, t: (b * MTN + t, 0))
    full = lambda a: pl.BlockSpec(a.shape, lambda b, p, t: (0,) * a.ndim)
    return pl.pallas_call(
        _pos_body,
        grid=(B, 3, MTN),
        in_specs=[rowsK(DIM), rowsM(16)] + [full(a) for a in params_t],
        out_specs=pl.BlockSpec((1, MTA, DIM), lambda b, p, t: (p, b * MTN + t, 0)),
        out_shape=jax.ShapeDtypeStruct((3, SR, DIM), F32),
        scratch_shapes=[pltpu.VMEM((8, 128), F32)],
    )(mo_g, smisc, *params_t)


CNT_D = (HID // NG) * M


def _final_body(dn_ref, anc_ref, pos_ref, ident_ref,
                de1_ref, de1b_ref, gn5w_ref, gn5b_ref, de2_ref, de2b_ref,
                f1_ref, f2_ref, f3_ref, fb_ref, out_ref, dn_out_ref):
    dn = dn_ref[:, 0:1] + dn_ref[:, 1:2]
    dn_out_ref[...] = dn
    h = jnp.dot(dn, de1_ref[...], preferred_element_type=F32) + de1b_ref[...]
    g = _gmat()
    e = _emat()
    s = jnp.dot(jnp.sum(h, axis=0, keepdims=True), g, preferred_element_type=F32)
    sq = jnp.dot(jnp.sum(h * h, axis=0, keepdims=True), g, preferred_element_type=F32)
    mean = s / CNT_D
    var = sq / CNT_D - mean * mean
    rstd = 1.0 / jnp.sqrt(var + 1e-5)
    ml = jnp.dot(mean, e, preferred_element_type=F32)
    rl = jnp.dot(rstd, e, preferred_element_type=F32)
    hn = jnp.maximum((h - ml) * rl * gn5w_ref[...] + gn5b_ref[...], 0.0)
    dens = jnp.dot(hn, de2_ref[...], preferred_element_type=F32) + de2b_ref[...]
    agg = (jnp.dot(anc_ref[...], f1_ref[...], preferred_element_type=F32)
           + jnp.dot(pos_ref[...], f2_ref[...], preferred_element_type=F32)
           + jnp.dot(dens, f3_ref[...], preferred_element_type=F32)
           + fb_ref[...])
    out_ref[...] = agg + ident_ref[...]


def _final(dn2t, anc, pos, ident, params_t):
    rowsM = lambda c: pl.BlockSpec((M, c), lambda b: (b, 0))
    full = lambda a: pl.BlockSpec(a.shape, lambda b: (0,) * a.ndim)
    return pl.pallas_call(
        _final_body,
        grid=(B,),
        in_specs=[rowsM(2), rowsM(DIM), rowsM(DIM), rowsM(DIM)]
        + [full(a) for a in params_t],
        out_specs=[rowsM(DIM), rowsM(1)],
        out_shape=[
            jax.ShapeDtypeStruct((SR, DIM), F32),
            jax.ShapeDtypeStruct((SR, 1), F32),
        ],
    )(dn2t, anc, pos, ident, *params_t)


def kernel(xyzs, feats, params):
    p = params
    x_bn = xyzs[:, 0, :]
    y_bn = xyzs[:, 1, :]
    z_bn = xyzs[:, 2, :]

    sample_idx, sx, sy, sz = _fps(x_bn, y_bn, z_bn)

    o2s = _o2s(x_bn.reshape(B * N, 1), y_bn.reshape(B * N, 1),
               z_bn.reshape(B * N, 1), sx, sy, sz).reshape(B, N)
    knn_i = _knn(sx.reshape(SR, 1), sy.reshape(SR, 1), sz.reshape(SR, 1),
                 x_bn, y_bn, z_bn)

    feats_rows = feats.transpose(0, 2, 1).reshape(B * N, DIM)
    f_rows, kv_rows = _convs(
        feats_rows,
        p['pre_w'].T, p['pre_b'].reshape(1, DIM),
        jnp.concatenate([p['wk'].T, p['wv'].T], axis=1),
        jnp.concatenate([p['bk'], p['bv']]).reshape(1, DIM),
    )

    boffN = (jnp.arange(B, dtype=I32) * N).reshape(B, 1)
    kidx2 = (knn_i.reshape(B, M * K) + boffN).reshape(KR // 128, 128)
    sidx2 = (sample_idx + boffN).reshape(SR // 128, 128)
    oidx2 = (o2s + (jnp.arange(B, dtype=I32) * M).reshape(B, 1)).reshape(OR // 128, 128)
    t_mo = jnp.concatenate(
        [jnp.stack([x_bn, y_bn, z_bn, o2s.astype(F32)], axis=-1).reshape(B * N, 4),
         jnp.zeros((B * N, DIM - 4), F32)], axis=1)

    kv_g, mo_g, sfeat_g, ident_g, dn2 = _sc_gathers(
        kv_rows, t_mo, f_rows, feats_rows, kidx2, sidx2, oidx2)
    dn2t = dn2.T

    smisc = jnp.concatenate(
        [jnp.stack([sx, sy, sz], axis=-1).reshape(SR, 3),
         jnp.zeros((SR, 13), F32)], axis=1)

    attn_params = (
        p['wq'].T, p['bq'].reshape(1, HID),
        jnp.concatenate([p['cd1_w'].T, jnp.zeros((13, HID), F32)], axis=0),
        p['cd1_b'].reshape(1, HID),
        p['cd_gn_w'].reshape(1, HID), p['cd_gn_b'].reshape(1, HID),
        p['cd2_w'].T, p['cd2_b'].reshape(1, HID),
        p['cg1_w'].T, p['cg1_b'].reshape(1, HID),
        p['cg_gn_w'].reshape(1, HID), p['cg_gn_b'].reshape(1, HID),
        p['cg2_w'].T, p['cg2_b'].reshape(1, HID),
        p['post_w'].T, p['post_b'].reshape(1, DIM),
    )
    anc3, md3 = _attn(kv_g, mo_g, sfeat_g, smisc, dn2t, attn_params)
    anc, md = anc3[2], md3[2]

    pos_params = (
        jnp.concatenate([p['pe1_w'].T, jnp.zeros((12, HID), F32)], axis=0),
        p['pe1_b'].reshape(1, HID),
        p['pe_gn_w'].reshape(1, HID), p['pe_gn_b'].reshape(1, HID),
        p['pe2_w'].T, p['pe2_b'].reshape(1, DIM),
        p['pa1_w'].T, p['pa1_b'].reshape(1, HID),
        p['pa_gn_w'].reshape(1, HID), p['pa_gn_b'].reshape(1, HID),
        p['pa2_w'].T, p['pa2_b'].reshape(1, DIM),
    )
    pos_e = _pos_emb(mo_g, smisc, pos_params)[2]

    fin_params = (
        p['de1_w'].T, p['de1_b'].reshape(1, HID),
        p['de_gn_w'].reshape(1, HID), p['de_gn_b'].reshape(1, HID),
        p['de2_w'].T, p['de2_b'].reshape(1, DIM),
        p['fin_w'][:, :DIM].T, p['fin_w'][:, DIM:2 * DIM].T,
        p['fin_w'][:, 2 * DIM:].T, p['fin_b'].reshape(1, DIM),
    )
    out_rows, dn_col = _final(dn2t, anc, pos_e, ident_g, fin_params)

    sampled_xyzs = jnp.stack([sx, sy, sz], axis=1)
    sampled_out = out_rows.reshape(B, M, DIM).transpose(0, 2, 1)
    downsample_num = dn_col.reshape(B, M)
    mean_distance = md.reshape(B, M)
    return (sampled_xyzs, sampled_out, downsample_num, mean_distance)


# speedup vs baseline: 18.5109x; 1.0769x over previous
import functools
import math

import jax
import jax.numpy as jnp
from jax import lax
from jax.experimental import pallas as pl
from jax.experimental.pallas import tpu as pltpu
from jax.experimental.pallas import tpu_sc as plsc

B = 4
N = 8192
DIM = 128
HID = 64
NG = 8
K = 16
M = 2048

NEG = -3.4028235e38
F32 = jnp.float32
I32 = jnp.int32

NC = 2
NS = 16
NW = NC * NS


FPS_S = N // 128
FPM_S = M // 128


def _fps_body(x_ref, y_ref, z_ref, si_ref, sx_ref, sy_ref, sz_ref):
    x = x_ref[...]
    y = y_ref[...]
    z = z_ref[...]
    ion = (lax.broadcasted_iota(I32, (B, FPS_S, 128), 1) * 128
           + lax.broadcasted_iota(I32, (B, FPS_S, 128), 2))
    iom = (lax.broadcasted_iota(I32, (B, FPM_S, 128), 1) * 128
           + lax.broadcasted_iota(I32, (B, FPM_S, 128), 2))

    def body(i, carry):
        dists, far = carry
        sel = ion == far
        cx = jnp.sum(jnp.where(sel, x, 0.0), axis=(1, 2), keepdims=True)
        cy = jnp.sum(jnp.where(sel, y, 0.0), axis=(1, 2), keepdims=True)
        cz = jnp.sum(jnp.where(sel, z, 0.0), axis=(1, 2), keepdims=True)
        lm = iom == i
        si_ref[...] = jnp.where(lm, jnp.broadcast_to(far, (B, FPM_S, 128)), si_ref[...])
        sx_ref[...] = jnp.where(lm, jnp.broadcast_to(cx, (B, FPM_S, 128)), sx_ref[...])
        sy_ref[...] = jnp.where(lm, jnp.broadcast_to(cy, (B, FPM_S, 128)), sy_ref[...])
        sz_ref[...] = jnp.where(lm, jnp.broadcast_to(cz, (B, FPM_S, 128)), sz_ref[...])
        dx = x - cx
        dy = y - cy
        dz = z - cz
        d = dx * dx + dy * dy + dz * dz
        dists = jnp.minimum(dists, d)
        mx = jnp.max(dists, axis=(1, 2), keepdims=True)
        far2 = jnp.min(jnp.where(dists == mx, ion, N), axis=(1, 2), keepdims=True)
        return dists, far2.astype(I32)

    d0 = jnp.full((B, FPS_S, 128), 1e10, F32)
    f0 = jnp.zeros((B, 1, 1), I32)
    lax.fori_loop(0, M, body, (d0, f0))


def _fps(x_bn, y_bn, z_bn):
    outs = pl.pallas_call(
        _fps_body,
        out_shape=[
            jax.ShapeDtypeStruct((B, FPM_S, 128), I32),
            jax.ShapeDtypeStruct((B, FPM_S, 128), F32),
            jax.ShapeDtypeStruct((B, FPM_S, 128), F32),
            jax.ShapeDtypeStruct((B, FPM_S, 128), F32),
        ],
    )(x_bn.reshape(B, FPS_S, 128), y_bn.reshape(B, FPS_S, 128),
      z_bn.reshape(B, FPS_S, 128))
    return [o.reshape(B, M) for o in outs]


NT1 = 512


def _o2s_body(xc_ref, yc_ref, zc_ref, sx_ref, sy_ref, sz_ref, o_ref):
    t = pl.program_id(0)
    b = t // (N // NT1)
    subl = lax.broadcasted_iota(I32, (B, M), 0)

    def pick(r):
        return jnp.sum(jnp.where(subl == b, r[...], 0.0), axis=0, keepdims=True)

    sx = pick(sx_ref)
    sy = pick(sy_ref)
    sz = pick(sz_ref)
    qx = xc_ref[...]
    qy = yc_ref[...]
    qz = zc_ref[...]
    bf = lambda v: v.astype(jnp.bfloat16).astype(F32)
    dot = bf(qx) * bf(sx) + bf(qy) * bf(sy) + bf(qz) * bf(sz)
    qq = qx * qx + qy * qy + qz * qz
    ss = sx * sx + sy * sy + sz * sz
    d = (qq + ss) - 2.0 * dot
    mn = jnp.min(d, axis=1, keepdims=True)
    iom = lax.broadcasted_iota(I32, (NT1, M), 1)
    idx = jnp.min(jnp.where(d == mn, iom, M), axis=1, keepdims=True)
    o_ref[...] = idx


def _o2s(x_c, y_c, z_c, sx, sy, sz):
    col = pl.BlockSpec((NT1, 1), lambda t: (t, 0))
    full = pl.BlockSpec((B, M), lambda t: (0, 0))
    return pl.pallas_call(
        _o2s_body,
        grid=(B * N // NT1,),
        in_specs=[col, col, col, full, full, full],
        out_specs=pl.BlockSpec((NT1, 1), lambda t: (t, 0)),
        out_shape=jax.ShapeDtypeStruct((B * N, 1), I32),
    )(x_c, y_c, z_c, sx, sy, sz)


MT2 = 256


def _knn_body(sx_ref, sy_ref, sz_ref, x_ref, y_ref, z_ref, o_ref):
    t = pl.program_id(0)
    b = t // (M // MT2)
    subl = lax.broadcasted_iota(I32, (B, N), 0)

    def pick(r):
        return jnp.sum(jnp.where(subl == b, r[...], 0.0), axis=0, keepdims=True)

    qx = sx_ref[...]
    qy = sy_ref[...]
    qz = sz_ref[...]
    x = pick(x_ref)
    y = pick(y_ref)
    z = pick(z_ref)
    bf = lambda v: v.astype(jnp.bfloat16).astype(F32)
    dot = bf(qx) * bf(x) + bf(qy) * bf(y) + bf(qz) * bf(z)
    qq = qx * qx + qy * qy + qz * qz
    ss = x * x + y * y + z * z
    d = (qq + ss) - 2.0 * dot
    ion = lax.broadcasted_iota(I32, (MT2, N), 1)
    for k in range(K):
        mn = jnp.min(d, axis=1, keepdims=True)
        idx = jnp.min(jnp.where(d == mn, ion, N), axis=1, keepdims=True)
        o_ref[:, :, k : k + 1] = idx.reshape(1, MT2, 1)
        d = jnp.where(ion == idx, jnp.inf, d)


def _knn(sx_c, sy_c, sz_c, x_bn, y_bn, z_bn):
    col = pl.BlockSpec((MT2, 1), lambda t: (t, 0))
    full = pl.BlockSpec((B, N), lambda t: (0, 0))
    tpb = M // MT2
    return pl.pallas_call(
        _knn_body,
        grid=(B * M // MT2,),
        in_specs=[col, col, col, full, full, full],
        out_specs=pl.BlockSpec((1, MT2, K), lambda t: (t // tpb, t % tpb, 0)),
        out_shape=jax.ShapeDtypeStruct((B, M, K), I32),
    )(sx_c, sy_c, sz_c, x_bn, y_bn, z_bn)


RT = 2048


def _convs_body(x_ref, wp_ref, bp_ref, wkv_ref, bkv_ref, f_ref, kv_ref):
    x = x_ref[...]
    f = jnp.dot(x, wp_ref[...], preferred_element_type=F32) + bp_ref[...]
    f_ref[...] = f
    kv_ref[...] = jnp.dot(f, wkv_ref[...], preferred_element_type=F32) + bkv_ref[...]


def _convs(feats_rows, wpT, bp, wkvT, bkv):
    full = lambda a: pl.BlockSpec(a.shape, lambda t: (0,) * a.ndim)
    return pl.pallas_call(
        _convs_body,
        grid=(B * N // RT,),
        in_specs=[
            pl.BlockSpec((RT, DIM), lambda t: (t, 0)),
            full(wpT), full(bp), full(wkvT), full(bkv),
        ],
        out_specs=[
            pl.BlockSpec((RT, DIM), lambda t: (t, 0)),
            pl.BlockSpec((RT, DIM), lambda t: (t, 0)),
        ],
        out_shape=[
            jax.ShapeDtypeStruct((B * N, DIM), F32),
            jax.ShapeDtypeStruct((B * N, DIM), F32),
        ],
    )(feats_rows, wpT, bp, wkvT, bkv)


KR = B * M * K
SR = B * M
OR = B * N
KR_W = KR // NW
SR_W = SR // NW
OR_W = OR // NW


def _sc_body(tkv, tmo, tf, tfeat, kidx2, sidx2, oidx2,
             kv_g, mo_g, sfeat_g, ident_g, dn2,
             idx_v, sidx_v, oidx_v, buf_kv, buf_mo,
             buf_f, buf_ft, ones_v, zb, shared, sem):
    cid = lax.axis_index("c")
    sid = lax.axis_index("s")
    wid = sid * NC + cid

    pltpu.sync_copy(kidx2.at[pl.ds(wid * 32, 32)], idx_v)
    pltpu.sync_copy(sidx2.at[pl.ds(wid * 2, 2)], sidx_v)
    pltpu.sync_copy(oidx2.at[pl.ds(wid * 8, 8)], oidx_v)

    for i in range(8):
        zb[pl.ds(i * 16, 16)] = jnp.zeros((16,), F32)
        ones_v[pl.ds(i * 16, 16)] = jnp.ones((16,), F32)

    for j in range(4):
        pltpu.sync_copy(zb, shared.at[pl.ds(sid * 512 + j * 128, 128)])
    plsc.subcore_barrier()

    def obody(j, c):
        pltpu.sync_copy(ones_v, shared.at[oidx_v.at[j]], add=True)
        return c

    lax.fori_loop(0, 8, obody, 0)
    plsc.subcore_barrier()

    @pl.when(sid == 0)
    def _():
        pltpu.sync_copy(shared, dn2.at[cid])

    def gbody(j, c):
        row = idx_v.at[j]
        base = wid * KR_W + j * 128
        pltpu.async_copy(tkv.at[row], buf_kv, sem).wait()
        pltpu.sync_copy(buf_kv, kv_g.at[pl.ds(base, 128)])
        pltpu.async_copy(tmo.at[row], buf_mo, sem).wait()
        pltpu.sync_copy(buf_mo, mo_g.at[pl.ds(base, 128)])
        return c

    lax.fori_loop(0, 32, gbody, 0)

    def sbody(j, c):
        row = sidx_v.at[j]
        base = wid * SR_W + j * 128
        pltpu.async_copy(tf.at[row], buf_f, sem).wait()
        pltpu.sync_copy(buf_f, sfeat_g.at[pl.ds(base, 128)])
        pltpu.async_copy(tfeat.at[row], buf_ft, sem).wait()
        pltpu.sync_copy(buf_ft, ident_g.at[pl.ds(base, 128)])
        return c

    lax.fori_loop(0, 2, sbody, 0)


def _sc_gathers(tkv, tmo, tf, tfeat, kidx2, sidx2, oidx2):
    mesh = plsc.VectorSubcoreMesh(core_axis_name="c", subcore_axis_name="s")
    return pl.kernel(
        _sc_body,
        out_type=[
            jax.ShapeDtypeStruct((KR, DIM), F32),
            jax.ShapeDtypeStruct((KR, DIM), F32),
            jax.ShapeDtypeStruct((SR, DIM), F32),
            jax.ShapeDtypeStruct((SR, DIM), F32),
            jax.ShapeDtypeStruct((NC, SR), F32),
        ],
        mesh=mesh,
        compiler_params=pltpu.CompilerParams(needs_layout_passes=False),
        scratch_types=[
            pltpu.VMEM((32, 128), I32),
            pltpu.VMEM((2, 128), I32),
            pltpu.VMEM((8, 128), I32),
            pltpu.VMEM((128, DIM), F32),
            pltpu.VMEM((128, DIM), F32),
            pltpu.VMEM((128, DIM), F32),
            pltpu.VMEM((128, DIM), F32),
            pltpu.VMEM((128,), F32),
            pltpu.VMEM((128,), F32),
            pltpu.VMEM_SHARED((SR,), F32),
            pltpu.SemaphoreType.DMA,
        ],
    )(tkv, tmo, tf, tfeat, kidx2, sidx2, oidx2)


def _gmat():
    ci = lax.broadcasted_iota(I32, (HID, NG), 0) // (HID // NG)
    gi = lax.broadcasted_iota(I32, (HID, NG), 1)
    return (ci == gi).astype(F32)


def _emat():
    ci = lax.broadcasted_iota(I32, (NG, HID), 1) // (HID // NG)
    gi = lax.broadcasted_iota(I32, (NG, HID), 0)
    return (ci == gi).astype(F32)


def _gn_accum(stats_ref, r0, h):
    g = _gmat()
    s = jnp.dot(jnp.sum(h, axis=0, keepdims=True), g, preferred_element_type=F32)
    sq = jnp.dot(jnp.sum(h * h, axis=0, keepdims=True), g, preferred_element_type=F32)
    stats_ref[r0 : r0 + 1, 0:NG] += s
    stats_ref[r0 + 1 : r0 + 2, 0:NG] += sq


def _gn_apply(stats_ref, r0, cnt, h, gam_ref, bet_ref):
    e = _emat()
    s = stats_ref[r0 : r0 + 1, 0:NG]
    sq = stats_ref[r0 + 1 : r0 + 2, 0:NG]
    mean = s / cnt
    var = sq / cnt - mean * mean
    rstd = 1.0 / jnp.sqrt(var + 1e-5)
    ml = jnp.dot(mean, e, preferred_element_type=F32)
    rl = jnp.dot(rstd, e, preferred_element_type=F32)
    return (h - ml) * rl * gam_ref[...] + bet_ref[...]


MTA = 256
MTN = M // MTA
CNT_A = (HID // NG) * M * K


def _attn_body(kv_ref, mo_ref, sfeat_ref, smisc_ref,
               dn_ref, wq_ref, bq_ref, cd1_ref, cd1b_ref, gn1w_ref, gn1b_ref,
               cd2_ref, cd2b_ref, cg1_ref, cg1b_ref, gn2w_ref, gn2b_ref,
               cg2_ref, cg2b_ref, post_ref, postb_ref,
               anc_ref, md_ref, stats_ref):
    p = pl.program_id(1)
    t = pl.program_id(2)

    @pl.when((p == 0) & (t == 0))
    def _():
        stats_ref[...] = jnp.zeros_like(stats_ref)

    mo = mo_ref[...]
    misc = mo[:, 0:16]
    smisc = smisc_ref[...]
    s3 = jnp.broadcast_to(smisc[:, None, :], (MTA, K, 16)).reshape(MTA * K, 16)
    pos = s3 - misc
    h1 = jnp.dot(pos, cd1_ref[...], preferred_element_type=F32) + cd1b_ref[...]

    @pl.when(p == 0)
    def _():
        _gn_accum(stats_ref, 0, h1)

    @pl.when(p > 0)
    def _():
        h1n = jnp.maximum(_gn_apply(stats_ref, 0, CNT_A, h1, gn1w_ref, gn1b_ref), 0.0)
        pos_enc = jnp.dot(h1n, cd2_ref[...], preferred_element_type=F32) + cd2b_ref[...]
        q = jnp.dot(sfeat_ref[...], wq_ref[...], preferred_element_type=F32) + bq_ref[...]
        kv = kv_ref[...]
        key3 = kv[:, 0:HID].reshape(MTA, K, HID)
        pe3 = pos_enc.reshape(MTA, K, HID)
        a0 = (q[:, None, :] - key3 + pe3).reshape(MTA * K, HID)
        a1 = jnp.dot(a0, cg1_ref[...], preferred_element_type=F32) + cg1b_ref[...]

        @pl.when(p == 1)
        def _():
            _gn_accum(stats_ref, 2, a1)

        @pl.when(p == 2)
        def _():
            a1n = jnp.maximum(_gn_apply(stats_ref, 2, CNT_A, a1, gn2w_ref, gn2b_ref), 0.0)
            a2 = (jnp.dot(a1n, cg2_ref[...], preferred_element_type=F32) + cg2b_ref[...])
            a2 = a2 * (1.0 / math.sqrt(HID))
            rowm = t * MTA + lax.div(lax.broadcasted_iota(I32, (MTA * K, 1), 0), K)
            maskr = mo[:, 3:4] == rowm.astype(F32)
            a3 = jnp.where(maskr, a2, NEG).reshape(MTA, K, HID)
            mx = jnp.max(a3, axis=1, keepdims=True)
            ex = jnp.exp(a3 - mx)
            soft = ex / jnp.sum(ex, axis=1, keepdims=True)
            v3 = kv[:, HID:DIM].reshape(MTA, K, HID) + pe3
            res = jnp.sum(soft * v3, axis=1)
            anc_ref[...] = (jnp.dot(res, post_ref[...], preferred_element_type=F32)
                            + postb_ref[...] + sfeat_ref[...])[None]
            lane16 = lax.broadcasted_iota(I32, (MTA * K, 16), 1)
            diff = jnp.where(lane16 < 3, misc - s3, 0.0)
            sq = jnp.sum(diff * diff, axis=1, keepdims=True) + 1e-12
            dist = jnp.sqrt(sq)
            dsum = jnp.sum(jnp.where(maskr, dist, 0.0).reshape(MTA, K, 1), axis=1)
            dn = dn_ref[:, 0:1] + dn_ref[:, 1:2]
            md_ref[...] = (dsum / dn)[None]


def _attn(kv_g, mo_g, sfeat_g, smisc, dn2t, params_t):
    (wqT, bq, cd1T, cd1b, gn1w, gn1b, cd2T, cd2b, cg1T, cg1b, gn2w, gn2b,
     cg2T, cg2b, postT, postb) = params_t
    rowsK = lambda c: pl.BlockSpec((MTA * K, c), lambda b, p, t: (b * MTN + t, 0))
    rowsM = lambda c: pl.BlockSpec((MTA, c), lambda b, p, t: (b * MTN + t, 0))
    full = lambda a: pl.BlockSpec(a.shape, lambda b, p, t: (0,) * a.ndim)
    return pl.pallas_call(
        _attn_body,
        grid=(B, 3, MTN),
        in_specs=[
            rowsK(DIM), rowsK(DIM), rowsM(DIM), rowsM(16), rowsM(2),
            full(wqT), full(bq), full(cd1T), full(cd1b), full(gn1w), full(gn1b),
            full(cd2T), full(cd2b), full(cg1T), full(cg1b), full(gn2w), full(gn2b),
            full(cg2T), full(cg2b), full(postT), full(postb),
        ],
        out_specs=[
            pl.BlockSpec((1, MTA, DIM), lambda b, p, t: (p, b * MTN + t, 0)),
            pl.BlockSpec((1, MTA, 1), lambda b, p, t: (p, b * MTN + t, 0)),
        ],
        out_shape=[
            jax.ShapeDtypeStruct((3, SR, DIM), F32),
            jax.ShapeDtypeStruct((3, SR, 1), F32),
        ],
        scratch_shapes=[pltpu.VMEM((8, 128), F32)],
    )(kv_g, mo_g, sfeat_g, smisc, dn2t, *params_t)


def _pos_body(mo_ref, smisc_ref,
              pe1_ref, pe1b_ref, gn3w_ref, gn3b_ref, pe2_ref, pe2b_ref,
              pa1_ref, pa1b_ref, gn4w_ref, gn4b_ref, pa2_ref, pa2b_ref,
              out_ref, stats_ref):
    p = pl.program_id(1)
    t = pl.program_id(2)

    @pl.when((p == 0) & (t == 0))
    def _():
        stats_ref[...] = jnp.zeros_like(stats_ref)

    mo = mo_ref[...]
    misc = mo[:, 0:16]
    smisc = smisc_ref[...]
    s3 = jnp.broadcast_to(smisc[:, None, :], (MTA, K, 16)).reshape(MTA * K, 16)
    lane = lax.broadcasted_iota(I32, (MTA * K, 16), 1)
    diff = jnp.where(lane < 3, misc - s3, 0.0)
    sq = jnp.sum(diff * diff, axis=1, keepdims=True) + 1e-12
    d = jnp.sqrt(sq)
    direction = diff / jnp.maximum(d, 1e-12)
    local = direction + jnp.where(lane == 3, d, 0.0)
    h = jnp.dot(local, pe1_ref[...], preferred_element_type=F32) + pe1b_ref[...]

    @pl.when(p == 0)
    def _():
        _gn_accum(stats_ref, 0, h)

    @pl.when(p > 0)
    def _():
        hn = jnp.maximum(_gn_apply(stats_ref, 0, CNT_A, h, gn3w_ref, gn3b_ref), 0.0)
        pe = jnp.dot(hn, pe2_ref[...], preferred_element_type=F32) + pe2b_ref[...]
        a2p = jnp.dot(pe, pa1_ref[...], preferred_element_type=F32) + pa1b_ref[...]

        @pl.when(p == 1)
        def _():
            _gn_accum(stats_ref, 2, a2p)

        @pl.when(p == 2)
        def _():
            a2n = jnp.maximum(_gn_apply(stats_ref, 2, CNT_A, a2p, gn4w_ref, gn4b_ref), 0.0)
            a2 = jnp.dot(a2n, pa2_ref[...], preferred_element_type=F32) + pa2b_ref[...]
            rowm = t * MTA + lax.div(lax.broadcasted_iota(I32, (MTA * K, 1), 0), K)
            maskr = mo[:, 3:4] == rowm.astype(F32)
            a3 = jnp.where(maskr, a2, NEG).reshape(MTA, K, DIM)
            mx = jnp.max(a3, axis=1, keepdims=True)
            ex = jnp.exp(a3 - mx)
            soft = ex / jnp.sum(ex, axis=1, keepdims=True)
            pe3 = pe.reshape(MTA, K, DIM)
            out_ref[...] = jnp.sum(soft * pe3, axis=1)[None]


def _pos_emb(mo_g, smisc, params_t):
    rowsK = lambda c: pl.BlockSpec((MTA * K, c), lambda b, p, t: (b * MTN + t, 0))
    rowsM = lambda c: pl.BlockSpec((MTA, c), lambda b, p, t: (b * MTN + t, 0))
    full = lambda a: pl.BlockSpec(a.shape, lambda b, p, t: (0,) * a.ndim)
    return pl.pallas_call(
        _pos_body,
        grid=(B, 3, MTN),
        in_specs=[rowsK(DIM), rowsM(16)] + [full(a) for a in params_t],
        out_specs=pl.BlockSpec((1, MTA, DIM), lambda b, p, t: (p, b * MTN + t, 0)),
        out_shape=jax.ShapeDtypeStruct((3, SR, DIM), F32),
        scratch_shapes=[pltpu.VMEM((8, 128), F32)],
    )(mo_g, smisc, *params_t)


CNT_D = (HID // NG) * M


def _final_body(dn_ref, anc_ref, pos_ref, ident_ref,
                de1_ref, de1b_ref, gn5w_ref, gn5b_ref, de2_ref, de2b_ref,
                f1_ref, f2_ref, f3_ref, fb_ref, out_ref, dn_out_ref):
    dn = dn_ref[:, 0:1] + dn_ref[:, 1:2]
    dn_out_ref[...] = dn
    h = jnp.dot(dn, de1_ref[...], preferred_element_type=F32) + de1b_ref[...]
    g = _gmat()
    e = _emat()
    s = jnp.dot(jnp.sum(h, axis=0, keepdims=True), g, preferred_element_type=F32)
    sq = jnp.dot(jnp.sum(h * h, axis=0, keepdims=True), g, preferred_element_type=F32)
    mean = s / CNT_D
    var = sq / CNT_D - mean * mean
    rstd = 1.0 / jnp.sqrt(var + 1e-5)
    ml = jnp.dot(mean, e, preferred_element_type=F32)
    rl = jnp.dot(rstd, e, preferred_element_type=F32)
    hn = jnp.maximum((h - ml) * rl * gn5w_ref[...] + gn5b_ref[...], 0.0)
    dens = jnp.dot(hn, de2_ref[...], preferred_element_type=F32) + de2b_ref[...]
    agg = (jnp.dot(anc_ref[...], f1_ref[...], preferred_element_type=F32)
           + jnp.dot(pos_ref[...], f2_ref[...], preferred_element_type=F32)
           + jnp.dot(dens, f3_ref[...], preferred_element_type=F32)
           + fb_ref[...])
    out_ref[...] = agg + ident_ref[...]


def _final(dn2t, anc, pos, ident, params_t):
    rowsM = lambda c: pl.BlockSpec((M, c), lambda b: (b, 0))
    full = lambda a: pl.BlockSpec(a.shape, lambda b: (0,) * a.ndim)
    return pl.pallas_call(
        _final_body,
        grid=(B,),
        in_specs=[rowsM(2), rowsM(DIM), rowsM(DIM), rowsM(DIM)]
        + [full(a) for a in params_t],
        out_specs=[rowsM(DIM), rowsM(1)],
        out_shape=[
            jax.ShapeDtypeStruct((SR, DIM), F32),
            jax.ShapeDtypeStruct((SR, 1), F32),
        ],
    )(dn2t, anc, pos, ident, *params_t)


def kernel(xyzs, feats, params):
    p = params
    x_bn = xyzs[:, 0, :]
    y_bn = xyzs[:, 1, :]
    z_bn = xyzs[:, 2, :]

    sample_idx, sx, sy, sz = _fps(x_bn, y_bn, z_bn)

    o2s = _o2s(x_bn.reshape(B * N, 1), y_bn.reshape(B * N, 1),
               z_bn.reshape(B * N, 1), sx, sy, sz).reshape(B, N)
    knn_i = _knn(sx.reshape(SR, 1), sy.reshape(SR, 1), sz.reshape(SR, 1),
                 x_bn, y_bn, z_bn)

    feats_rows = feats.transpose(0, 2, 1).reshape(B * N, DIM)
    f_rows, kv_rows = _convs(
        feats_rows,
        p['pre_w'].T, p['pre_b'].reshape(1, DIM),
        jnp.concatenate([p['wk'].T, p['wv'].T], axis=1),
        jnp.concatenate([p['bk'], p['bv']]).reshape(1, DIM),
    )

    boffN = (jnp.arange(B, dtype=I32) * N).reshape(B, 1)
    kidx2 = (knn_i.reshape(B, M * K) + boffN).reshape(KR // 128, 128)
    sidx2 = (sample_idx + boffN).reshape(SR // 128, 128)
    oidx2 = (o2s + (jnp.arange(B, dtype=I32) * M).reshape(B, 1)).reshape(OR // 128, 128)
    t_mo = jnp.concatenate(
        [jnp.stack([x_bn, y_bn, z_bn, o2s.astype(F32)], axis=-1).reshape(B * N, 4),
         jnp.zeros((B * N, DIM - 4), F32)], axis=1)

    kv_g, mo_g, sfeat_g, ident_g, dn2 = _sc_gathers(
        kv_rows, t_mo, f_rows, feats_rows, kidx2, sidx2, oidx2)
    dn2t = dn2.T

    smisc = jnp.concatenate(
        [jnp.stack([sx, sy, sz], axis=-1).reshape(SR, 3),
         jnp.zeros((SR, 13), F32)], axis=1)

    attn_params = (
        p['wq'].T, p['bq'].reshape(1, HID),
        jnp.concatenate([p['cd1_w'].T, jnp.zeros((13, HID), F32)], axis=0),
        p['cd1_b'].reshape(1, HID),
        p['cd_gn_w'].reshape(1, HID), p['cd_gn_b'].reshape(1, HID),
        p['cd2_w'].T, p['cd2_b'].reshape(1, HID),
        p['cg1_w'].T, p['cg1_b'].reshape(1, HID),
        p['cg_gn_w'].reshape(1, HID), p['cg_gn_b'].reshape(1, HID),
        p['cg2_w'].T, p['cg2_b'].reshape(1, HID),
        p['post_w'].T, p['post_b'].reshape(1, DIM),
    )
    anc3, md3 = _attn(kv_g, mo_g, sfeat_g, smisc, dn2t, attn_params)
    anc, md = anc3[2], md3[2]

    pos_params = (
        jnp.concatenate([p['pe1_w'].T, jnp.zeros((12, HID), F32)], axis=0),
        p['pe1_b'].reshape(1, HID),
        p['pe_gn_w'].reshape(1, HID), p['pe_gn_b'].reshape(1, HID),
        p['pe2_w'].T, p['pe2_b'].reshape(1, DIM),
        p['pa1_w'].T, p['pa1_b'].reshape(1, HID),
        p['pa_gn_w'].reshape(1, HID), p['pa_gn_b'].reshape(1, HID),
        p['pa2_w'].T, p['pa2_b'].reshape(1, DIM),
    )
    pos_e = _pos_emb(mo_g, smisc, pos_params)[2]

    fin_params = (
        p['de1_w'].T, p['de1_b'].reshape(1, HID),
        p['de_gn_w'].reshape(1, HID), p['de_gn_b'].reshape(1, HID),
        p['de2_w'].T, p['de2_b'].reshape(1, DIM),
        p['fin_w'][:, :DIM].T, p['fin_w'][:, DIM:2 * DIM].T,
        p['fin_w'][:, 2 * DIM:].T, p['fin_b'].reshape(1, DIM),
    )
    out_rows, dn_col = _final(dn2t, anc, pos_e, ident_g, fin_params)

    sampled_xyzs = jnp.stack([sx, sy, sz], axis=1)
    sampled_out = out_rows.reshape(B, M, DIM).transpose(0, 2, 1)
    downsample_num = dn_col.reshape(B, M)
    mean_distance = md.reshape(B, M)
    return (sampled_xyzs, sampled_out, downsample_num, mean_distance)
